# speedup vs baseline: 5.6412x; 5.6412x over previous
import functools

import jax
import jax.numpy as jnp
from jax.experimental import pallas as pl
from jax.experimental.pallas import tpu as pltpu

BLK = 2048


def _body(feat_ref, batch_ref, vn_ref, w_ref, b_ref, out_ref, vnout_ref,
          acc_ref, vn_scr, *, n_rows, num_graphs):
    p = pl.program_id(0)
    i = pl.program_id(1)

    bvec = batch_ref[0, 0, :]
    row0 = i * BLK

    @pl.when(p == 0)
    def _pool_phase():
        @pl.when(i == 0)
        def _init():
            acc_ref[...] = jnp.zeros_like(acc_ref)

        gids = jax.lax.broadcasted_iota(jnp.int32, (num_graphs, BLK), 0)
        rids = jax.lax.broadcasted_iota(jnp.int32, (num_graphs, BLK), 1) + row0
        onehot_t = jnp.where((gids == bvec[None, :]) & (rids < n_rows), 1.0, 0.0)
        rmask = (jax.lax.broadcasted_iota(jnp.int32, (BLK, 1), 0) + row0) < n_rows
        featv = jnp.where(rmask, feat_ref[...], 0.0)
        acc_ref[...] += jnp.dot(onehot_t, featv,
                                preferred_element_type=jnp.float32)

    @pl.when(p == 1)
    def _bcast_phase():
        @pl.when(i == 0)
        def _fc():
            vn_tmp = acc_ref[...] + vn_ref[...]
            vn_o = jnp.maximum(
                jnp.dot(vn_tmp, w_ref[...], preferred_element_type=jnp.float32)
                + b_ref[...], 0.0) + vn_ref[...]
            vn_scr[...] = vn_o
            vnout_ref[...] = vn_o

        gids = jax.lax.broadcasted_iota(jnp.int32, (BLK, num_graphs), 1)
        onehot = jnp.where(gids == bvec[:, None], 1.0, 0.0)
        out_ref[...] = feat_ref[...] + jnp.dot(
            onehot, vn_scr[...], preferred_element_type=jnp.float32)


def kernel(feat, vn_feat, W, b, batch):
    n, d = feat.shape
    num_graphs = vn_feat.shape[0]
    num_blocks = (n + BLK - 1) // BLK
    pad = num_blocks * BLK - n
    batch_r = jnp.pad(batch, (0, pad)).reshape(num_blocks, 1, BLK)

    grid = (2, num_blocks)
    out_shape = (
        jax.ShapeDtypeStruct((n, d), jnp.float32),
        jax.ShapeDtypeStruct((num_graphs, d), jnp.float32),
    )
    feat_out, vn_out = pl.pallas_call(
        functools.partial(_body, n_rows=n, num_graphs=num_graphs),
        grid=grid,
        in_specs=[
            pl.BlockSpec((BLK, d), lambda p, i: (i, 0)),
            pl.BlockSpec((1, 1, BLK), lambda p, i: (i, 0, 0)),
            pl.BlockSpec((num_graphs, d), lambda p, i: (0, 0)),
            pl.BlockSpec((d, d), lambda p, i: (0, 0)),
            pl.BlockSpec((1, d), lambda p, i: (0, 0)),
        ],
        out_specs=[
            pl.BlockSpec((BLK, d), lambda p, i: (jnp.where(p == 1, i, 0), 0)),
            pl.BlockSpec((num_graphs, d), lambda p, i: (0, 0)),
        ],
        scratch_shapes=[
            pltpu.VMEM((num_graphs, d), jnp.float32),
            pltpu.VMEM((num_graphs, d), jnp.float32),
        ],
        out_shape=out_shape,
        compiler_params=pltpu.CompilerParams(
            dimension_semantics=("arbitrary", "arbitrary"),
        ),
    )(feat, batch_r, vn_feat, W, b.reshape(1, d))
    return (feat_out, vn_out)
